# baseline (speedup 1.0000x reference)
import functools

import jax
import jax.numpy as jnp
from jax import lax
from jax.experimental import pallas as pl
from jax.experimental.pallas import tpu as pltpu
from jax.experimental.pallas import tpu_sc as plsc

NC = 2
NS = 16
NW = NC * NS
CHUNK = 128
NBUF = 4
PDIST = 2
WAVE = 16


def _sc_mesh():
    return plsc.VectorSubcoreMesh(core_axis_name="c", subcore_axis_name="s")


def _make_deg_kernel(chunks, npad, rows_per_tile):
    @functools.partial(
        pl.kernel,
        out_type=jax.ShapeDtypeStruct((NC, npad, 16), jnp.float32),
        mesh=_sc_mesh(),
        compiler_params=pltpu.CompilerParams(use_tc_tiling_on_sc=False),
        scratch_types=[
            pltpu.VMEM((chunks, CHUNK), jnp.int32),
            pltpu.VMEM((CHUNK, 16), jnp.float32),
            pltpu.VMEM_SHARED((npad, 16), jnp.float32),
            pltpu.SemaphoreType.DMA,
        ],
    )
    def deg_kernel(dst_hbm, ones_hbm, zeros_hbm, out_hbm, dst_v, ones_v, deg_sh,
                   sem):
        c = lax.axis_index("c")
        s = lax.axis_index("s")
        base = s * rows_per_tile
        rows = pl.ds(base, rows_per_tile)
        pltpu.sync_copy(dst_hbm.at[c, s], dst_v)
        pltpu.sync_copy(ones_hbm, ones_v)
        pltpu.sync_copy(zeros_hbm.at[rows], deg_sh.at[rows])
        plsc.subcore_barrier()

        def body(i, carry):
            for b in range(WAVE):
                j = i * WAVE + b
                pltpu.async_copy(ones_v, deg_sh.at[dst_v.at[j]], sem, add=True)
            for b in range(WAVE):
                j = i * WAVE + b
                pltpu.make_async_copy(ones_v, deg_sh.at[dst_v.at[j]], sem).wait()
            return carry

        lax.fori_loop(0, chunks // WAVE, body, 0)
        plsc.subcore_barrier()
        pltpu.sync_copy(deg_sh.at[rows], out_hbm.at[c, rows])

    return deg_kernel


def _make_msg_kernel(chunks, npad, rows_per_tile, d_hid):
    @functools.partial(
        pl.kernel,
        out_type=jax.ShapeDtypeStruct((NC, npad, d_hid), jnp.float32),
        mesh=_sc_mesh(),
        compiler_params=pltpu.CompilerParams(use_tc_tiling_on_sc=False),
        scratch_types=[
            pltpu.VMEM((chunks, CHUNK), jnp.int32),
            pltpu.VMEM((chunks, CHUNK), jnp.int32),
            pltpu.VMEM((NBUF, CHUNK, d_hid), jnp.float32),
            pltpu.VMEM_SHARED((npad, d_hid), jnp.float32),
            pltpu.SemaphoreType.DMA((NBUF,)),
            pltpu.SemaphoreType.DMA((NBUF,)),
        ],
    )
    def msg_kernel(src_hbm, dst_hbm, g_hbm, zeros_hbm, out_hbm,
                   src_v, dst_v, rows_v, acc_sh, gsem, ssem):
        c = lax.axis_index("c")
        s = lax.axis_index("s")
        base = s * rows_per_tile
        rows = pl.ds(base, rows_per_tile)
        pltpu.sync_copy(src_hbm.at[c, s], src_v)
        pltpu.sync_copy(dst_hbm.at[c, s], dst_v)
        pltpu.sync_copy(zeros_hbm.at[rows], acc_sh.at[rows])
        plsc.subcore_barrier()

        def gather(j, b):
            return pltpu.make_async_copy(
                g_hbm.at[src_v.at[j]], rows_v.at[b], gsem.at[b])

        def scatter(j, b):
            return pltpu.make_async_copy(
                rows_v.at[b], acc_sh.at[dst_v.at[j]], ssem.at[b])

        for r in range(PDIST):
            gather(r, r).start()

        nblk = chunks // NBUF

        def emit(i, head=False, tail=False):
            for b in range(NBUF):
                j = i * NBUF + b
                gather(j, b).wait()
                pltpu.async_copy(rows_v.at[b], acc_sh.at[dst_v.at[j]],
                                 ssem.at[b], add=True)
                if tail and b >= PDIST:
                    continue
                r = j + PDIST
                br = (b + PDIST) % NBUF
                if not (head and b < NBUF - PDIST):
                    scatter(r - NBUF, br).wait()
                gather(r, br).start()

        emit(0, head=True)
        lax.fori_loop(1, nblk - 1, lambda i, cy: (emit(i), cy)[1], 0)
        emit(nblk - 1, tail=True)
        for b in range(NBUF):
            scatter(chunks - NBUF + b, b).wait()
        plsc.subcore_barrier()
        pltpu.sync_copy(acc_sh.at[rows], out_hbm.at[c, rows])

    return msg_kernel


def _matmul_body(x_ref, w_ref, h_ref):
    h_ref[...] = jnp.dot(x_ref[...], w_ref[...],
                         preferred_element_type=jnp.float32)


def _scale_body(h_ref, degp_ref, g_ref):
    deg = degp_ref[0] + degp_ref[1]
    dinv = lax.rsqrt(deg[:, 0:1] + 1.0)
    g = h_ref[...] * dinv
    npad = g_ref.shape[0]
    n = g.shape[0]
    g_ref[...] = jnp.concatenate(
        [g, jnp.zeros((npad - n, g.shape[1]), g.dtype)], axis=0)


def _head_body(accp_ref, g_ref, degp_ref, bc_ref, wlt_ref, bl_ref, out_ref):
    deg = degp_ref[0] + degp_ref[1]
    dinv = lax.rsqrt(deg[:, 0:1] + 1.0)
    z = (accp_ref[0] + accp_ref[1] + g_ref[...]) * dinv + bc_ref[...]
    zr = jnp.maximum(z, 0.0)
    o = jnp.sum(zr * wlt_ref[...], axis=1, keepdims=True) + bl_ref[...]
    out_ref[...] = jax.nn.sigmoid(o)


def kernel(x, edge_index, W_conv, b_conv, W_lin, b_lin):
    n = x.shape[0]
    d_in = x.shape[1]
    d_hid = W_conv.shape[1]
    e = edge_index.shape[1]

    rows_per_tile = pl.cdiv(n, NS * 8) * 8
    npad = rows_per_tile * NS
    chunks = pl.cdiv(pl.cdiv(e, NW), CHUNK * WAVE) * WAVE
    epad = NW * chunks * CHUNK
    src = edge_index[0].astype(jnp.int32)
    dst = edge_index[1].astype(jnp.int32)
    pad = n + jnp.arange(epad - e, dtype=jnp.int32) % (npad - n)
    src4 = jnp.concatenate([src, pad]).reshape(NC, NS, chunks, CHUNK)
    dst4 = jnp.concatenate([dst, pad]).reshape(NC, NS, chunks, CHUNK)

    ones16 = jnp.ones((CHUNK, 16), jnp.float32)
    zeros16 = jnp.zeros((npad, 16), jnp.float32)
    zeros_hid = jnp.zeros((npad, d_hid), jnp.float32)

    h = pl.pallas_call(
        _matmul_body,
        grid=(1,),
        in_specs=[
            pl.BlockSpec((n, d_in), lambda i: (0, 0)),
            pl.BlockSpec((d_in, d_hid), lambda i: (0, 0)),
        ],
        out_specs=pl.BlockSpec((n, d_hid), lambda i: (0, 0)),
        out_shape=jax.ShapeDtypeStruct((n, d_hid), jnp.float32),
    )(x, W_conv)

    degp = _make_deg_kernel(chunks, npad, rows_per_tile)(dst4, ones16, zeros16)

    g = pl.pallas_call(
        _scale_body,
        grid=(1,),
        in_specs=[
            pl.BlockSpec((n, d_hid), lambda i: (0, 0)),
            pl.BlockSpec((NC, n, 16), lambda i: (0, 0, 0)),
        ],
        out_specs=pl.BlockSpec((npad, d_hid), lambda i: (0, 0)),
        out_shape=jax.ShapeDtypeStruct((npad, d_hid), jnp.float32),
    )(h, degp)

    accp = _make_msg_kernel(chunks, npad, rows_per_tile, d_hid)(
        src4, dst4, g, zeros_hid)

    out = pl.pallas_call(
        _head_body,
        grid=(1,),
        in_specs=[
            pl.BlockSpec((NC, n, d_hid), lambda i: (0, 0, 0)),
            pl.BlockSpec((n, d_hid), lambda i: (0, 0)),
            pl.BlockSpec((NC, n, 16), lambda i: (0, 0, 0)),
            pl.BlockSpec((1, d_hid), lambda i: (0, 0)),
            pl.BlockSpec((1, d_hid), lambda i: (0, 0)),
            pl.BlockSpec((1, 1), lambda i: (0, 0)),
        ],
        out_specs=pl.BlockSpec((n, 1), lambda i: (0, 0)),
        out_shape=jax.ShapeDtypeStruct((n, 1), jnp.float32),
    )(accp, g, degp, b_conv.reshape(1, d_hid), W_lin.reshape(1, d_hid),
      b_lin.reshape(1, 1))

    return out


# speedup vs baseline: 1.1364x; 1.1364x over previous
import functools

import jax
import jax.numpy as jnp
from jax import lax
from jax.experimental import pallas as pl
from jax.experimental.pallas import tpu as pltpu
from jax.experimental.pallas import tpu_sc as plsc

NC = 2
NS = 16
NW = NC * NS
CHUNK = 128
NBUF = 4
WAVE = 8


def _sc_mesh():
    return plsc.VectorSubcoreMesh(core_axis_name="c", subcore_axis_name="s")


def _make_deg_kernel(chunks, npad, rows_per_tile):
    @functools.partial(
        pl.kernel,
        out_type=jax.ShapeDtypeStruct((NC, npad, 16), jnp.float32),
        mesh=_sc_mesh(),
        compiler_params=pltpu.CompilerParams(use_tc_tiling_on_sc=False),
        scratch_types=[
            pltpu.VMEM((chunks, CHUNK), jnp.int32),
            pltpu.VMEM((CHUNK, 16), jnp.float32),
            pltpu.VMEM_SHARED((npad, 16), jnp.float32),
            pltpu.SemaphoreType.DMA,
        ],
    )
    def deg_kernel(dst_hbm, ones_hbm, zeros_hbm, out_hbm, dst_v, ones_v, deg_sh,
                   sem):
        c = lax.axis_index("c")
        s = lax.axis_index("s")
        base = s * rows_per_tile
        rows = pl.ds(base, rows_per_tile)
        pltpu.sync_copy(dst_hbm.at[c, s], dst_v)
        pltpu.sync_copy(ones_hbm, ones_v)
        pltpu.sync_copy(zeros_hbm.at[rows], deg_sh.at[rows])
        plsc.subcore_barrier()

        def body(i, carry):
            for b in range(WAVE):
                j = i * WAVE + b
                pltpu.async_copy(ones_v, deg_sh.at[dst_v.at[j]], sem, add=True)
            for b in range(WAVE):
                j = i * WAVE + b
                pltpu.make_async_copy(ones_v, deg_sh.at[dst_v.at[j]], sem).wait()
            return carry

        lax.fori_loop(0, chunks // WAVE, body, 0)
        plsc.subcore_barrier()
        pltpu.sync_copy(deg_sh.at[rows], out_hbm.at[c, rows])

    return deg_kernel


def _make_msg_kernel(chunks, npad, rows_per_tile, d_hid):
    @functools.partial(
        pl.kernel,
        out_type=jax.ShapeDtypeStruct((NC, npad, d_hid), jnp.float32),
        mesh=_sc_mesh(),
        compiler_params=pltpu.CompilerParams(use_tc_tiling_on_sc=False),
        scratch_types=[
            pltpu.VMEM((chunks, CHUNK), jnp.int32),
            pltpu.VMEM((chunks, CHUNK), jnp.int32),
            pltpu.VMEM((NBUF, CHUNK, d_hid), jnp.float32),
            pltpu.VMEM_SHARED((npad, d_hid), jnp.float32),
            pltpu.SemaphoreType.DMA((NBUF,)),
            pltpu.SemaphoreType.DMA((NBUF,)),
        ],
    )
    def msg_kernel(src_hbm, dst_hbm, g_hbm, zeros_hbm, out_hbm,
                   src_v, dst_v, rows_v, acc_sh, gsem, ssem):
        c = lax.axis_index("c")
        s = lax.axis_index("s")
        base = s * rows_per_tile
        rows = pl.ds(base, rows_per_tile)
        pltpu.sync_copy(src_hbm.at[c, s], src_v)
        pltpu.sync_copy(dst_hbm.at[c, s], dst_v)
        pltpu.sync_copy(zeros_hbm.at[rows], acc_sh.at[rows])
        plsc.subcore_barrier()

        def gather(j, b):
            return pltpu.make_async_copy(
                g_hbm.at[src_v.at[j]], rows_v.at[b], gsem.at[b])

        def scatter(j, b):
            return pltpu.make_async_copy(
                rows_v.at[b], acc_sh.at[dst_v.at[j]], ssem.at[b])

        for b in range(NBUF):
            gather(b, b).start()

        def body(i, carry):
            for b in range(NBUF):
                j = i * NBUF + b
                gather(j, b).wait()
                pltpu.async_copy(rows_v.at[b], acc_sh.at[dst_v.at[j]],
                                 ssem.at[b], add=True)

                @pl.when(j + NBUF < chunks)
                def _():
                    scatter(j, b).wait()
                    gather(j + NBUF, b).start()

            return carry

        lax.fori_loop(0, chunks // NBUF, body, 0)
        for b in range(NBUF):
            scatter(chunks - NBUF + b, b).wait()
        plsc.subcore_barrier()
        pltpu.sync_copy(acc_sh.at[rows], out_hbm.at[c, rows])

    return msg_kernel


def _matmul_body(x_ref, w_ref, h_ref):
    h_ref[...] = jnp.dot(x_ref[...], w_ref[...],
                         preferred_element_type=jnp.float32)


def _scale_body(h_ref, degp_ref, g_ref):
    deg = degp_ref[0] + degp_ref[1]
    dinv = lax.rsqrt(deg[:, 0:1] + 1.0)
    g = h_ref[...] * dinv
    npad = g_ref.shape[0]
    n = g.shape[0]
    g_ref[...] = jnp.concatenate(
        [g, jnp.zeros((npad - n, g.shape[1]), g.dtype)], axis=0)


def _head_body(accp_ref, g_ref, degp_ref, bc_ref, wlt_ref, bl_ref, out_ref):
    deg = degp_ref[0] + degp_ref[1]
    dinv = lax.rsqrt(deg[:, 0:1] + 1.0)
    z = (accp_ref[0] + accp_ref[1] + g_ref[...]) * dinv + bc_ref[...]
    zr = jnp.maximum(z, 0.0)
    o = jnp.sum(zr * wlt_ref[...], axis=1, keepdims=True) + bl_ref[...]
    out_ref[...] = jax.nn.sigmoid(o)


def kernel(x, edge_index, W_conv, b_conv, W_lin, b_lin):
    n = x.shape[0]
    d_in = x.shape[1]
    d_hid = W_conv.shape[1]
    e = edge_index.shape[1]

    rows_per_tile = pl.cdiv(n, NS * 8) * 8
    npad = rows_per_tile * NS
    chunks = pl.cdiv(pl.cdiv(e, NW), CHUNK * WAVE) * WAVE
    epad = NW * chunks * CHUNK
    src = edge_index[0].astype(jnp.int32)
    dst = edge_index[1].astype(jnp.int32)
    pad = n + jnp.arange(epad - e, dtype=jnp.int32) % (npad - n)
    src4 = jnp.concatenate([src, pad]).reshape(NC, NS, chunks, CHUNK)
    dst4 = jnp.concatenate([dst, pad]).reshape(NC, NS, chunks, CHUNK)

    ones16 = jnp.ones((CHUNK, 16), jnp.float32)
    zeros16 = jnp.zeros((npad, 16), jnp.float32)
    zeros_hid = jnp.zeros((npad, d_hid), jnp.float32)

    h = pl.pallas_call(
        _matmul_body,
        grid=(1,),
        in_specs=[
            pl.BlockSpec((n, d_in), lambda i: (0, 0)),
            pl.BlockSpec((d_in, d_hid), lambda i: (0, 0)),
        ],
        out_specs=pl.BlockSpec((n, d_hid), lambda i: (0, 0)),
        out_shape=jax.ShapeDtypeStruct((n, d_hid), jnp.float32),
    )(x, W_conv)

    degp = _make_deg_kernel(chunks, npad, rows_per_tile)(dst4, ones16, zeros16)

    g = pl.pallas_call(
        _scale_body,
        grid=(1,),
        in_specs=[
            pl.BlockSpec((n, d_hid), lambda i: (0, 0)),
            pl.BlockSpec((NC, n, 16), lambda i: (0, 0, 0)),
        ],
        out_specs=pl.BlockSpec((npad, d_hid), lambda i: (0, 0)),
        out_shape=jax.ShapeDtypeStruct((npad, d_hid), jnp.float32),
    )(h, degp)

    accp = _make_msg_kernel(chunks, npad, rows_per_tile, d_hid)(
        src4, dst4, g, zeros_hid)

    out = pl.pallas_call(
        _head_body,
        grid=(1,),
        in_specs=[
            pl.BlockSpec((NC, n, d_hid), lambda i: (0, 0, 0)),
            pl.BlockSpec((n, d_hid), lambda i: (0, 0)),
            pl.BlockSpec((NC, n, 16), lambda i: (0, 0, 0)),
            pl.BlockSpec((1, d_hid), lambda i: (0, 0)),
            pl.BlockSpec((1, d_hid), lambda i: (0, 0)),
            pl.BlockSpec((1, 1), lambda i: (0, 0)),
        ],
        out_specs=pl.BlockSpec((n, 1), lambda i: (0, 0)),
        out_shape=jax.ShapeDtypeStruct((n, 1), jnp.float32),
    )(accp, g, degp, b_conv.reshape(1, d_hid), W_lin.reshape(1, d_hid),
      b_lin.reshape(1, 1))

    return out
